# Initial kernel scaffold; baseline (speedup 1.0000x reference)
import jax
import jax.numpy as jnp
from jax.experimental import pallas as pl


def kernel(pred_boxes, pred_scores, true_boxes, true_classes, pboxes):
    raise NotImplementedError("write your pallas kernel here")


# speedup vs baseline: 3.4484x; 3.4484x over previous
import functools

import jax
import jax.numpy as jnp
from jax import lax
from jax.experimental import pallas as pl

_BS = 32
_NP = 8732
_NO = 16
_NC = 81


def _match_body(tb_ref, tc_ref, pbx_ref, pbc_ref, pb_ref,
                lab_ref, stats_ref):
    tb = tb_ref[0]
    ox1 = tb[:, 0:1]
    oy1 = tb[:, 1:2]
    ox2 = tb[:, 2:3]
    oy2 = tb[:, 3:4]
    px1 = pbx_ref[0:1, :]
    py1 = pbx_ref[1:2, :]
    px2 = pbx_ref[2:3, :]
    py2 = pbx_ref[3:4, :]

    w = jnp.maximum(jnp.minimum(ox2, px2) - jnp.maximum(ox1, px1), 0.0)
    h = jnp.maximum(jnp.minimum(oy2, py2) - jnp.maximum(oy1, py1), 0.0)
    inter = w * h
    area_o = (ox2 - ox1) * (oy2 - oy1)
    area_p = (px2 - px1) * (py2 - py1)
    iou = inter / (area_o + area_p - inter)

    jidx = lax.broadcasted_iota(jnp.int32, (_NO, _NP), 0)
    pidx = lax.broadcasted_iota(jnp.int32, (_NO, _NP), 1)

    col_max = jnp.max(iou, axis=0, keepdims=True)
    obj_idx = jnp.min(jnp.where(iou == col_max, jidx, _NO),
                      axis=0, keepdims=True)
    row_max = jnp.max(iou, axis=1, keepdims=True)
    best_p = jnp.min(jnp.where(iou == row_max, pidx, _NP),
                     axis=1, keepdims=True)

    match = pidx == best_p
    forced_j = jnp.max(jnp.where(match, jidx, -1), axis=0, keepdims=True)
    obj_idx = jnp.where(forced_j >= 0, forced_j, obj_idx)
    ov = jnp.where(forced_j >= 0, 1.0, col_max)

    onehot = obj_idx == jidx
    tc_col = tc_ref[0]
    labels = jnp.sum(jnp.where(onehot, tc_col, 0), axis=0, keepdims=True)
    labels = jnp.where(ov < 0.5, 0, labels)
    lab_ref[0] = labels

    def gath(c):
        col = tb[:, c:c + 1]
        return jnp.sum(jnp.where(onehot, col, 0.0), axis=0, keepdims=True)

    gx1, gy1, gx2, gy2 = gath(0), gath(1), gath(2), gath(3)
    cx = (gx1 + gx2) * 0.5
    cy = (gy1 + gy2) * 0.5
    bw = gx2 - gx1
    bh = gy2 - gy1
    pcx = pbc_ref[0:1, :]
    pcy = pbc_ref[1:2, :]
    pw = pbc_ref[2:3, :]
    ph = pbc_ref[3:4, :]
    gcx = (cx - pcx) / (pw * 0.1)
    gcy = (cy - pcy) / (ph * 0.1)
    gw = jnp.log(bw / pw) * 5.0
    gh = jnp.log(bh / ph) * 5.0

    posf = (labels != 0).astype(jnp.float32)
    pb = pb_ref[0]
    locsum = (jnp.sum(jnp.abs(pb[0:1, :] - gcx) * posf)
              + jnp.sum(jnp.abs(pb[1:2, :] - gcy) * posf)
              + jnp.sum(jnp.abs(pb[2:3, :] - gw) * posf)
              + jnp.sum(jnp.abs(pb[3:4, :] - gh) * posf))
    npos = jnp.sum(posf)
    stats_ref[0, 0:1, :] = jnp.full((1, 128), locsum, jnp.float32)
    stats_ref[0, 1:2, :] = jnp.full((1, 128), npos, jnp.float32)


def _ce_body(sc_ref, lab_ref, cls_ref, cp_ref):
    s = sc_ref[0]
    m = jnp.max(s, axis=-1, keepdims=True)
    se = jnp.sum(jnp.exp(s - m), axis=-1, keepdims=True)
    lse = jnp.log(se) + m
    lab = lab_ref[0]
    cidx = lax.broadcasted_iota(jnp.int32, (_NP, _NC), 1)
    s_at = jnp.sum(jnp.where(cidx == lab, s, 0.0), axis=-1, keepdims=True)
    cls = lse - s_at
    cls_ref[0] = cls
    posf = (lab != 0).astype(jnp.float32)
    cp_ref[0, 0:1, :] = jnp.full((1, 128), jnp.sum(cls * posf), jnp.float32)


def _combine_body(cls_ref, stats_ref, cp_ref, out_ref):
    cls = cls_ref[...]
    stats = stats_ref[...]
    locsum = jnp.sum(stats[:, 0:1, 0:1])
    npos = jnp.sum(stats[:, 1:2, 0:1])
    clspos = jnp.sum(cp_ref[...][:, :, 0:1])

    k = jnp.minimum((3.0 * npos).astype(jnp.int32), _NP)
    bits = lax.bitcast_convert_type(cls, jnp.int32)

    def step(_, carry):
        lo, hi = carry
        mid = lo + ((hi - lo) >> 1)
        cnt = jnp.sum((bits >= mid).astype(jnp.int32), axis=1,
                      keepdims=True)
        ge = cnt >= k
        return jnp.where(ge, mid, lo), jnp.where(ge, hi, mid)

    lo0 = jnp.zeros((_BS, 1), jnp.int32)
    hi0 = jnp.full((_BS, 1), 0x7F800000, jnp.int32)
    lo, _ = lax.fori_loop(0, 31, step, (lo0, hi0))
    tval = lax.bitcast_convert_type(lo, jnp.float32)
    gt = bits > lo
    cnt_gt = jnp.sum(gt.astype(jnp.float32), axis=1, keepdims=True)
    sum_gt = jnp.sum(jnp.where(gt, cls, 0.0), axis=1, keepdims=True)
    topk = jnp.sum(sum_gt + (k.astype(jnp.float32) - cnt_gt) * tval)

    loss = locsum / (npos * 4.0) + (clspos + topk) / npos
    out_ref[...] = jnp.full((1, 1), loss, jnp.float32)


@jax.jit
def kernel(pred_boxes, pred_scores, true_boxes, true_classes, pboxes):
    f32 = jnp.float32
    pbc_t = pboxes.T
    pbx_t = jnp.concatenate([pbc_t[:2] - pbc_t[2:] / 2.0,
                             pbc_t[:2] + pbc_t[2:] / 2.0], axis=0)
    tc3 = true_classes.reshape(_BS, _NO, 1).astype(jnp.int32)
    pb_t = jnp.transpose(pred_boxes, (0, 2, 1))

    labels, stats = pl.pallas_call(
        _match_body,
        grid=(_BS,),
        in_specs=[
            pl.BlockSpec((1, _NO, 4), lambda i: (i, 0, 0)),
            pl.BlockSpec((1, _NO, 1), lambda i: (i, 0, 0)),
            pl.BlockSpec((4, _NP), lambda i: (0, 0)),
            pl.BlockSpec((4, _NP), lambda i: (0, 0)),
            pl.BlockSpec((1, 4, _NP), lambda i: (i, 0, 0)),
        ],
        out_specs=[
            pl.BlockSpec((1, 1, _NP), lambda i: (i, 0, 0)),
            pl.BlockSpec((1, 2, 128), lambda i: (i, 0, 0)),
        ],
        out_shape=[
            jax.ShapeDtypeStruct((_BS, 1, _NP), jnp.int32),
            jax.ShapeDtypeStruct((_BS, 2, 128), f32),
        ],
    )(true_boxes, tc3, pbx_t, pbc_t, pb_t)

    lab_b = labels.reshape(_BS, _NP, 1)
    cls_b, clspos = pl.pallas_call(
        _ce_body,
        grid=(_BS,),
        in_specs=[
            pl.BlockSpec((1, _NP, _NC), lambda i: (i, 0, 0)),
            pl.BlockSpec((1, _NP, 1), lambda i: (i, 0, 0)),
        ],
        out_specs=[
            pl.BlockSpec((1, _NP, 1), lambda i: (i, 0, 0)),
            pl.BlockSpec((1, 1, 128), lambda i: (i, 0, 0)),
        ],
        out_shape=[
            jax.ShapeDtypeStruct((_BS, _NP, 1), f32),
            jax.ShapeDtypeStruct((_BS, 1, 128), f32),
        ],
    )(pred_scores, lab_b)

    cls2 = cls_b.reshape(_BS, _NP)
    out = pl.pallas_call(
        _combine_body,
        out_shape=jax.ShapeDtypeStruct((1, 1), f32),
    )(cls2, stats, clspos)
    return out[0, 0]
